# baseline (speedup 1.0000x reference)
import functools

import jax
import jax.numpy as jnp
from jax import lax
from jax.experimental import pallas as pl
from jax.experimental.pallas import tpu as pltpu
from jax.experimental.pallas import tpu_sc as plsc

_EPS = 1e-6
_K_SC = 4
_NW = 32
_CHUNK = 8192


def _tc_sums_kernel(pred_ref, gt_text_ref, gt_kernels_ref, out_ref):
    gt = gt_text_ref[0, 0]
    pos = (gt > 0.5).astype(jnp.float32)

    rows = []
    for ch in range(6):
        s = jax.nn.sigmoid(pred_ref[0, ch])
        if ch == 0:
            t = gt
            m = pos
        else:
            t = gt_kernels_ref[0, ch - 1]
            m = None
        st = s * t
        ss = s * s
        tt = t * t
        if m is not None:
            st = st * m
            ss = ss * m
            tt = tt * m
        rows.append(jnp.sum(st, axis=0))
        rows.append(jnp.sum(ss, axis=0))
        rows.append(jnp.sum(tt, axis=0))
    rows.extend([jnp.zeros((512,), jnp.float32)] * 6)
    out_ref[0] = jnp.stack(rows, axis=0)


def _make_sc_sums(k_img, hw):
    n_units = k_img * 6
    out_len = n_units * 3 * 16
    mesh = plsc.VectorSubcoreMesh(core_axis_name="c", subcore_axis_name="s")

    rows_per_chunk = 16
    n_groups = 4
    nbuf = 4

    @functools.partial(
        pl.kernel,
        mesh=mesh,
        out_type=jax.ShapeDtypeStruct((_NW * out_len,), jnp.float32),
        scratch_types=(
            [pltpu.VMEM((rows_per_chunk, 512), jnp.float32)] * (2 * nbuf)
            + [pltpu.VMEM((out_len,), jnp.float32)]
            + [pltpu.SemaphoreType.DMA] * (2 * nbuf)
        ),
        compiler_params=pltpu.CompilerParams(use_tc_tiling_on_sc=True),
    )
    def sc_sums(pred_hbm, gt_text_hbm, gt_k_hbm, out_hbm, *scratch):
        iobufs = scratch[:2 * nbuf]
        obuf = scratch[2 * nbuf]
        sems = scratch[2 * nbuf + 1:]
        bufs = [(iobufs[2 * i], iobufs[2 * i + 1], sems[2 * i],
                 sems[2 * i + 1]) for i in range(nbuf)]
        wid = lax.axis_index("s") * 2 + lax.axis_index("c")
        base = wid * rows_per_chunk
        zero = jnp.zeros((16,), jnp.float32)
        chunks = [(b, ch) for b in range(k_img) for ch in range(6)]

        def issue(j):
            b, ch = chunks[j]
            pb, tb, ps, ts = bufs[j % nbuf]
            hp = pltpu.async_copy(
                pred_hbm.at[pl.ds((b * 6 + ch) * 512 + base, rows_per_chunk),
                            :], pb, ps)
            if ch == 0:
                src = gt_text_hbm.at[pl.ds(b * 512 + base, rows_per_chunk), :]
            else:
                src = gt_k_hbm.at[
                    pl.ds((b * 5 + ch - 1) * 512 + base, rows_per_chunk), :]
            ht = pltpu.async_copy(src, tb, ts)
            return hp, ht

        pending = [issue(j) for j in range(min(nbuf, len(chunks)))]
        for j, (b, ch) in enumerate(chunks):
            pb, tb, _, _ = bufs[j % nbuf]
            hp, ht = pending[j]
            hp.wait()
            ht.wait()

            def row_body(r, acc, ch=ch, pb=pb, tb=tb):
                def col_body(i, acc2, ch=ch, pb=pb, tb=tb):
                    new = []
                    for g in range(n_groups):
                        st, ss, tt = acc2[g]
                        col = (i * n_groups + g) * 16
                        x = pb[r, pl.ds(col, 16)]
                        t = tb[r, pl.ds(col, 16)]
                        s = 1.0 / (1.0 + jnp.exp(-x))
                        if ch == 0:
                            m = jnp.where(t > 0.5, 1.0, 0.0)
                            sm = s * m
                            tm = t * m
                            new.append((st + sm * t, ss + sm * s,
                                        tt + tm * t))
                        else:
                            new.append((st + s * t, ss + s * s, tt + t * t))
                    return tuple(new)

                return lax.fori_loop(0, 32 // n_groups, col_body, acc,
                                     unroll=2)

            acc0 = tuple((zero, zero, zero) for _ in range(n_groups))
            acc = lax.fori_loop(0, rows_per_chunk, row_body, acc0)
            st = acc[0][0] + acc[1][0] + acc[2][0] + acc[3][0]
            ss = acc[0][1] + acc[1][1] + acc[2][1] + acc[3][1]
            tt = acc[0][2] + acc[1][2] + acc[2][2] + acc[3][2]
            u = b * 6 + ch
            obuf[pl.ds((3 * u + 0) * 16, 16)] = st
            obuf[pl.ds((3 * u + 1) * 16, 16)] = ss
            obuf[pl.ds((3 * u + 2) * 16, 16)] = tt
            if j + nbuf < len(chunks):
                pending.append(issue(j + nbuf))
        pltpu.sync_copy(obuf, out_hbm.at[pl.ds(wid * out_len, out_len)])

    return sc_sums


def kernel(pred, gt_text, gt_kernels, training_mask):
    del training_mask
    b, c, h, w = pred.shape
    k = _K_SC

    sc_out = _make_sc_sums(k, h * w)(
        pred.reshape(b * c * h, w),
        gt_text.reshape(b * h, w),
        gt_kernels.reshape(b * (c - 1) * h, w),
    )

    tc_out = pl.pallas_call(
        _tc_sums_kernel,
        grid=(b - k,),
        in_specs=[
            pl.BlockSpec((1, c, h, w), lambda i: (i + k, 0, 0, 0)),
            pl.BlockSpec((1, 1, h, w), lambda i: (i + k, 0, 0, 0)),
            pl.BlockSpec((1, c - 1, h, w), lambda i: (i + k, 0, 0, 0)),
        ],
        out_specs=pl.BlockSpec((1, 24, w), lambda i: (i, 0, 0)),
        out_shape=jax.ShapeDtypeStruct((b - k, 24, w), jnp.float32),
    )(pred, gt_text, gt_kernels)

    sc_sums = sc_out.reshape(_NW, k * 6 * 3, 16).sum(axis=(0, 2)).reshape(
        k, 6, 3)
    tc_sums = tc_out[:, :18, :].sum(axis=-1).reshape(b - k, 6, 3)
    sums = jnp.concatenate([sc_sums, tc_sums], axis=0)

    inter, u1, u2 = sums[..., 0], sums[..., 1], sums[..., 2]
    dice = 1.0 - 2.0 * inter / (u1 + u2 + _EPS)
    loss_text = dice[:, 0].mean()
    loss_kernels = dice[:, 1:].mean()
    loss = loss_kernels + 0.5 * loss_text
    return (loss, loss_text, loss_kernels)


# speedup vs baseline: 1.0086x; 1.0086x over previous
import functools

import jax
import jax.numpy as jnp
from jax import lax
from jax.experimental import pallas as pl
from jax.experimental.pallas import tpu as pltpu
from jax.experimental.pallas import tpu_sc as plsc

_EPS = 1e-6
_K_SC = 2
_NW = 32
_CHUNK = 8192


def _tc_sums_kernel(pred_ref, gt_text_ref, gt_kernels_ref, out_ref):
    gt = gt_text_ref[0, 0]
    pos = (gt > 0.5).astype(jnp.float32)

    rows = []
    for ch in range(6):
        s = jax.nn.sigmoid(pred_ref[0, ch])
        if ch == 0:
            t = gt
            m = pos
        else:
            t = gt_kernels_ref[0, ch - 1]
            m = None
        st = s * t
        ss = s * s
        tt = t * t
        if m is not None:
            st = st * m
            ss = ss * m
            tt = tt * m
        rows.append(jnp.sum(st, axis=0))
        rows.append(jnp.sum(ss, axis=0))
        rows.append(jnp.sum(tt, axis=0))
    rows.extend([jnp.zeros((512,), jnp.float32)] * 6)
    out_ref[0] = jnp.stack(rows, axis=0)


def _make_sc_sums(k_img, hw):
    n_units = k_img * 6
    out_len = n_units * 3 * 16
    mesh = plsc.VectorSubcoreMesh(core_axis_name="c", subcore_axis_name="s")

    rows_per_chunk = 16
    n_groups = 4
    nbuf = 4

    @functools.partial(
        pl.kernel,
        mesh=mesh,
        out_type=jax.ShapeDtypeStruct((_NW * out_len,), jnp.float32),
        scratch_types=(
            [pltpu.VMEM((rows_per_chunk, 512), jnp.float32)] * (2 * nbuf)
            + [pltpu.VMEM((out_len,), jnp.float32)]
            + [pltpu.SemaphoreType.DMA] * (2 * nbuf)
        ),
        compiler_params=pltpu.CompilerParams(use_tc_tiling_on_sc=True),
    )
    def sc_sums(pred_hbm, gt_text_hbm, gt_k_hbm, out_hbm, *scratch):
        iobufs = scratch[:2 * nbuf]
        obuf = scratch[2 * nbuf]
        sems = scratch[2 * nbuf + 1:]
        bufs = [(iobufs[2 * i], iobufs[2 * i + 1], sems[2 * i],
                 sems[2 * i + 1]) for i in range(nbuf)]
        wid = lax.axis_index("s") * 2 + lax.axis_index("c")
        base = wid * rows_per_chunk
        zero = jnp.zeros((16,), jnp.float32)
        chunks = [(b, ch) for b in range(k_img) for ch in range(6)]

        def issue(j):
            b, ch = chunks[j]
            pb, tb, ps, ts = bufs[j % nbuf]
            hp = pltpu.async_copy(
                pred_hbm.at[pl.ds((b * 6 + ch) * 512 + base, rows_per_chunk),
                            :], pb, ps)
            if ch == 0:
                src = gt_text_hbm.at[pl.ds(b * 512 + base, rows_per_chunk), :]
            else:
                src = gt_k_hbm.at[
                    pl.ds((b * 5 + ch - 1) * 512 + base, rows_per_chunk), :]
            ht = pltpu.async_copy(src, tb, ts)
            return hp, ht

        pending = [issue(j) for j in range(min(nbuf, len(chunks)))]
        for j, (b, ch) in enumerate(chunks):
            pb, tb, _, _ = bufs[j % nbuf]
            hp, ht = pending[j]
            hp.wait()
            ht.wait()

            def row_body(r, acc, ch=ch, pb=pb, tb=tb):
                def col_body(i, acc2, ch=ch, pb=pb, tb=tb):
                    new = []
                    for g in range(n_groups):
                        st, ss, tt = acc2[g]
                        col = (i * n_groups + g) * 16
                        x = pb[r, pl.ds(col, 16)]
                        t = tb[r, pl.ds(col, 16)]
                        s = 1.0 / (1.0 + jnp.exp(-x))
                        if ch == 0:
                            m = jnp.where(t > 0.5, 1.0, 0.0)
                            sm = s * m
                            tm = t * m
                            new.append((st + sm * t, ss + sm * s,
                                        tt + tm * t))
                        else:
                            new.append((st + s * t, ss + s * s, tt + t * t))
                    return tuple(new)

                return lax.fori_loop(0, 32 // n_groups, col_body, acc,
                                     unroll=2)

            acc0 = tuple((zero, zero, zero) for _ in range(n_groups))
            acc = lax.fori_loop(0, rows_per_chunk, row_body, acc0)
            st = acc[0][0] + acc[1][0] + acc[2][0] + acc[3][0]
            ss = acc[0][1] + acc[1][1] + acc[2][1] + acc[3][1]
            tt = acc[0][2] + acc[1][2] + acc[2][2] + acc[3][2]
            u = b * 6 + ch
            obuf[pl.ds((3 * u + 0) * 16, 16)] = st
            obuf[pl.ds((3 * u + 1) * 16, 16)] = ss
            obuf[pl.ds((3 * u + 2) * 16, 16)] = tt
            if j + nbuf < len(chunks):
                pending.append(issue(j + nbuf))
        pltpu.sync_copy(obuf, out_hbm.at[pl.ds(wid * out_len, out_len)])

    return sc_sums


def kernel(pred, gt_text, gt_kernels, training_mask):
    del training_mask
    b, c, h, w = pred.shape
    k = _K_SC

    sc_out = _make_sc_sums(k, h * w)(
        pred.reshape(b * c * h, w),
        gt_text.reshape(b * h, w),
        gt_kernels.reshape(b * (c - 1) * h, w),
    )

    tc_out = pl.pallas_call(
        _tc_sums_kernel,
        grid=(b - k,),
        in_specs=[
            pl.BlockSpec((1, c, h, w), lambda i: (i + k, 0, 0, 0)),
            pl.BlockSpec((1, 1, h, w), lambda i: (i + k, 0, 0, 0)),
            pl.BlockSpec((1, c - 1, h, w), lambda i: (i + k, 0, 0, 0)),
        ],
        out_specs=pl.BlockSpec((1, 24, w), lambda i: (i, 0, 0)),
        out_shape=jax.ShapeDtypeStruct((b - k, 24, w), jnp.float32),
    )(pred, gt_text, gt_kernels)

    sc_sums = sc_out.reshape(_NW, k * 6 * 3, 16).sum(axis=(0, 2)).reshape(
        k, 6, 3)
    tc_sums = tc_out[:, :18, :].sum(axis=-1).reshape(b - k, 6, 3)
    sums = jnp.concatenate([sc_sums, tc_sums], axis=0)

    inter, u1, u2 = sums[..., 0], sums[..., 1], sums[..., 2]
    dice = 1.0 - 2.0 * inter / (u1 + u2 + _EPS)
    loss_text = dice[:, 0].mean()
    loss_kernels = dice[:, 1:].mean()
    loss = loss_kernels + 0.5 * loss_text
    return (loss, loss_text, loss_kernels)


# speedup vs baseline: 1.0285x; 1.0197x over previous
import functools

import jax
import jax.numpy as jnp
from jax import lax
from jax.experimental import pallas as pl
from jax.experimental.pallas import tpu as pltpu
from jax.experimental.pallas import tpu_sc as plsc

_EPS = 1e-6
_K_SC = 4
_NW = 32
_CHUNK = 8192


def _tc_sums_kernel(pred_ref, gt_text_ref, gt_kernels_ref, out_ref):
    gt = gt_text_ref[0, 0]
    pos = (gt > 0.5).astype(jnp.float32)

    rows = []
    for ch in range(6):
        s = jax.nn.sigmoid(pred_ref[0, ch])
        if ch == 0:
            t = gt
            m = pos
        else:
            t = gt_kernels_ref[0, ch - 1]
            m = None
        st = s * t
        ss = s * s
        tt = t * t
        if m is not None:
            st = st * m
            ss = ss * m
            tt = tt * m
        rows.append(jnp.sum(st, axis=0))
        rows.append(jnp.sum(ss, axis=0))
        rows.append(jnp.sum(tt, axis=0))
    rows.extend([jnp.zeros((512,), jnp.float32)] * 6)
    out_ref[0] = jnp.stack(rows, axis=0)


def _make_sc_sums(k_img, hw):
    n_units = k_img * 6
    out_len = n_units * 3 * 16
    mesh = plsc.VectorSubcoreMesh(core_axis_name="c", subcore_axis_name="s")

    rows_per_chunk = 16
    n_groups = 4
    nbuf = 2

    @functools.partial(
        pl.kernel,
        mesh=mesh,
        out_type=jax.ShapeDtypeStruct((_NW * out_len,), jnp.float32),
        scratch_types=(
            [pltpu.VMEM((rows_per_chunk, 512), jnp.float32)] * (2 * nbuf)
            + [pltpu.VMEM((out_len,), jnp.float32)]
            + [pltpu.SemaphoreType.DMA] * (2 * nbuf)
        ),
        compiler_params=pltpu.CompilerParams(use_tc_tiling_on_sc=True),
    )
    def sc_sums(pred_hbm, gt_text_hbm, gt_k_hbm, out_hbm, *scratch):
        iobufs = scratch[:2 * nbuf]
        obuf = scratch[2 * nbuf]
        sems = scratch[2 * nbuf + 1:]
        bufs = [(iobufs[2 * i], iobufs[2 * i + 1], sems[2 * i],
                 sems[2 * i + 1]) for i in range(nbuf)]
        wid = lax.axis_index("s") * 2 + lax.axis_index("c")
        base = wid * rows_per_chunk
        zero = jnp.zeros((16,), jnp.float32)

        def compute(pb, tb, masked):
            def row_body(r, acc):
                def col_body(i, acc2):
                    new = []
                    for g in range(n_groups):
                        st, ss, tt = acc2[g]
                        col = (i * n_groups + g) * 16
                        x = pb[r, pl.ds(col, 16)]
                        t = tb[r, pl.ds(col, 16)]
                        s = 1.0 / (1.0 + jnp.exp(-x))
                        if masked:
                            m = jnp.where(t > 0.5, 1.0, 0.0)
                            sm = s * m
                            tm = t * m
                            new.append((st + sm * t, ss + sm * s,
                                        tt + tm * t))
                        else:
                            new.append((st + s * t, ss + s * s, tt + t * t))
                    return tuple(new)

                return lax.fori_loop(0, 32 // n_groups, col_body, acc,
                                     unroll=2)

            acc0 = tuple((zero, zero, zero) for _ in range(n_groups))
            acc = lax.fori_loop(0, rows_per_chunk, row_body, acc0)
            st = acc[0][0] + acc[1][0] + acc[2][0] + acc[3][0]
            ss = acc[0][1] + acc[1][1] + acc[2][1] + acc[3][1]
            tt = acc[0][2] + acc[1][2] + acc[2][2] + acc[3][2]
            return st, ss, tt

        def store(unit, st, ss, tt):
            obuf[pl.ds((3 * unit + 0) * 16, 16)] = st
            obuf[pl.ds((3 * unit + 1) * 16, 16)] = ss
            obuf[pl.ds((3 * unit + 2) * 16, 16)] = tt

        def wait_pair(i):
            pb, tb, ps, ts = bufs[i]
            pltpu.make_async_copy(
                pred_hbm.at[pl.ds(0, rows_per_chunk), :], pb, ps).wait()
            pltpu.make_async_copy(
                pred_hbm.at[pl.ds(0, rows_per_chunk), :], tb, ts).wait()

        n_ku = k_img * 5

        def issue_k(u, i):
            pb, tb, ps, ts = bufs[i]
            b = u // 5
            ch5 = u - b * 5
            prow = (b * 6 + ch5 + 1) * 512 + base
            pltpu.async_copy(
                pred_hbm.at[pl.ds(prow, rows_per_chunk), :], pb, ps)
            pltpu.async_copy(
                gt_k_hbm.at[pl.ds(u * 512 + base, rows_per_chunk), :], tb, ts)

        for i in range(nbuf):
            issue_k(i, i)

        def k_body(j, _):
            for i in range(nbuf):
                u = j * nbuf + i
                wait_pair(i)
                pb, tb, _, _ = bufs[i]
                st, ss, tt = compute(pb, tb, masked=False)
                b = u // 5
                ch5 = u - b * 5
                store(b * 6 + ch5 + 1, st, ss, tt)

                @pl.when(u + nbuf < n_ku)
                def _():
                    issue_k(u + nbuf, i)
            return 0

        lax.fori_loop(0, n_ku // nbuf, k_body, 0)

        def issue_t(b, i):
            pb, tb, ps, ts = bufs[i]
            pltpu.async_copy(
                pred_hbm.at[pl.ds(b * 6 * 512 + base, rows_per_chunk), :],
                pb, ps)
            pltpu.async_copy(
                gt_text_hbm.at[pl.ds(b * 512 + base, rows_per_chunk), :],
                tb, ts)

        for i in range(min(nbuf, k_img)):
            issue_t(i, i)

        def t_body(j, _):
            for i in range(nbuf):
                b = j * nbuf + i
                wait_pair(i)
                pb, tb, _, _ = bufs[i]
                st, ss, tt = compute(pb, tb, masked=True)
                store(b * 6, st, ss, tt)

                @pl.when(b + nbuf < k_img)
                def _():
                    issue_t(b + nbuf, i)
            return 0

        lax.fori_loop(0, k_img // nbuf, t_body, 0)

        pltpu.sync_copy(obuf, out_hbm.at[pl.ds(wid * out_len, out_len)])

    return sc_sums


def kernel(pred, gt_text, gt_kernels, training_mask):
    del training_mask
    b, c, h, w = pred.shape
    k = _K_SC

    sc_out = _make_sc_sums(k, h * w)(
        pred.reshape(b * c * h, w),
        gt_text.reshape(b * h, w),
        gt_kernels.reshape(b * (c - 1) * h, w),
    )

    tc_out = pl.pallas_call(
        _tc_sums_kernel,
        grid=(b - k,),
        in_specs=[
            pl.BlockSpec((1, c, h, w), lambda i: (i + k, 0, 0, 0)),
            pl.BlockSpec((1, 1, h, w), lambda i: (i + k, 0, 0, 0)),
            pl.BlockSpec((1, c - 1, h, w), lambda i: (i + k, 0, 0, 0)),
        ],
        out_specs=pl.BlockSpec((1, 24, w), lambda i: (i, 0, 0)),
        out_shape=jax.ShapeDtypeStruct((b - k, 24, w), jnp.float32),
    )(pred, gt_text, gt_kernels)

    sc_sums = sc_out.reshape(_NW, k * 6 * 3, 16).sum(axis=(0, 2)).reshape(
        k, 6, 3)
    tc_sums = tc_out[:, :18, :].sum(axis=-1).reshape(b - k, 6, 3)
    sums = jnp.concatenate([sc_sums, tc_sums], axis=0)

    inter, u1, u2 = sums[..., 0], sums[..., 1], sums[..., 2]
    dice = 1.0 - 2.0 * inter / (u1 + u2 + _EPS)
    loss_text = dice[:, 0].mean()
    loss_kernels = dice[:, 1:].mean()
    loss = loss_kernels + 0.5 * loss_text
    return (loss, loss_text, loss_kernels)


# speedup vs baseline: 1.2407x; 1.2062x over previous
import jax
import jax.numpy as jnp
from jax.experimental import pallas as pl

_EPS = 1e-6


def _sums_kernel(pred_ref, gt_text_ref, gt_kernels_ref, out_ref):
    gt = gt_text_ref[0, 0]
    pos = (gt > 0.5).astype(jnp.float32)

    rows = []
    for ch in range(6):
        s = jax.nn.sigmoid(pred_ref[0, ch])
        if ch == 0:
            t = gt
            m = pos
        else:
            t = gt_kernels_ref[0, ch - 1]
            m = None
        st = s * t
        ss = s * s
        tt = t * t
        if m is not None:
            st = st * m
            ss = ss * m
            tt = tt * m
        rows.append(jnp.sum(st, axis=0))
        rows.append(jnp.sum(ss, axis=0))
        rows.append(jnp.sum(tt, axis=0))
    rows.extend([jnp.zeros((512,), jnp.float32)] * 6)
    out_ref[0] = jnp.stack(rows, axis=0)


def kernel(pred, gt_text, gt_kernels, training_mask):
    del training_mask
    b, c, h, w = pred.shape

    partials = pl.pallas_call(
        _sums_kernel,
        grid=(b,),
        in_specs=[
            pl.BlockSpec((1, c, h, w), lambda i: (i, 0, 0, 0)),
            pl.BlockSpec((1, 1, h, w), lambda i: (i, 0, 0, 0)),
            pl.BlockSpec((1, c - 1, h, w), lambda i: (i, 0, 0, 0)),
        ],
        out_specs=pl.BlockSpec((1, 24, w), lambda i: (i, 0, 0)),
        out_shape=jax.ShapeDtypeStruct((b, 24, w), jnp.float32),
    )(pred, gt_text, gt_kernels)

    sums = partials[:, :18, :].sum(axis=-1)
    inter = sums[:, 0::3]
    u1 = sums[:, 1::3]
    u2 = sums[:, 2::3]
    dice = 1.0 - 2.0 * inter / (u1 + u2 + _EPS)
    loss_text = dice[:, 0].mean()
    loss_kernels = dice[:, 1:].mean()
    loss = loss_kernels + 0.5 * loss_text
    return (loss, loss_text, loss_kernels)
